# baseline (speedup 1.0000x reference)
import jax
import jax.numpy as jnp
from jax import lax
from jax.experimental import pallas as pl
from jax.experimental.pallas import tpu as pltpu

_EPS = 1e-5


def _conv_stats_kernel(xp_ref, band_ref, y_ref, stats_ref):
    B, H, WCin = xp_ref.shape
    x = xp_ref[...].astype(jnp.bfloat16)
    z = jnp.zeros((B, 1, WCin), jnp.bfloat16)
    x = jnp.concatenate([z, x, z], axis=1)
    acc = jnp.dot(x[:, 0:H, :].reshape(B * H, WCin), band_ref[0],
                  preferred_element_type=jnp.float32)
    acc = acc + jnp.dot(x[:, 1:H + 1, :].reshape(B * H, WCin), band_ref[1],
                        preferred_element_type=jnp.float32)
    acc = acc + jnp.dot(x[:, 2:H + 2, :].reshape(B * H, WCin), band_ref[2],
                        preferred_element_type=jnp.float32)
    s1 = jnp.sum(acc, axis=0, keepdims=True)
    s2 = jnp.sum(acc * acc, axis=0, keepdims=True)
    stats_ref[0] = jnp.concatenate([s1, s2], axis=0)
    y_ref[...] = acc.reshape(B, H, -1).astype(jnp.bfloat16)


def _apply_kernel(y_ref, scale_ref, shift_ref, o_ref):
    y = y_ref[...].astype(jnp.float32)
    o_ref[...] = jnp.maximum(y * scale_ref[...] + shift_ref[...], 0.0)


def kernel(x_nchw, bands, b, gamma, beta, drop_mask_nc):
    del b
    N, Cin, H, W = x_nchw.shape
    WCin = W * Cin
    WCout = bands.shape[-1]
    Cout = WCout // W

    x_rows = jnp.transpose(x_nchw, (0, 2, 1, 3)).reshape(N, H, WCin)
    bands_bf = (bands.reshape(3, W, Cin, WCout)
                .transpose(0, 2, 1, 3)
                .reshape(3, WCin, WCout)
                .astype(jnp.bfloat16))

    cparams = pltpu.CompilerParams(
        dimension_semantics=("parallel",),
        vmem_limit_bytes=64 * 1024 * 1024,
    )

    b1 = 64 if N % 64 == 0 else 1
    g1 = N // b1
    y_rows, stats = pl.pallas_call(
        _conv_stats_kernel,
        grid=(g1,),
        in_specs=[
            pl.BlockSpec((b1, H, WCin), lambda n: (n, 0, 0)),
            pl.BlockSpec((3, WCin, WCout), lambda n: (0, 0, 0)),
        ],
        out_specs=[
            pl.BlockSpec((b1, H, WCout), lambda n: (n, 0, 0)),
            pl.BlockSpec((1, 2, WCout), lambda n: (n, 0, 0)),
        ],
        out_shape=[
            jax.ShapeDtypeStruct((N, H, WCout), jnp.bfloat16),
            jax.ShapeDtypeStruct((g1, 2, WCout), jnp.float32),
        ],
        compiler_params=cparams,
    )(x_rows, bands_bf)

    cnt = jnp.float32(N * H * W)
    tot = stats[:, 0, :].reshape(g1, W, Cout).sum(axis=(0, 1))
    tot_sq = stats[:, 1, :].reshape(g1, W, Cout).sum(axis=(0, 1))
    mean = tot / cnt
    var = jnp.maximum(tot_sq / cnt - mean * mean, 0.0)
    inv_std = lax.rsqrt(var + _EPS)

    scale_c = gamma * inv_std
    shift_c = beta - mean * scale_c
    scale_nc = scale_c[None, :] * drop_mask_nc
    shift_nc = shift_c[None, :] * drop_mask_nc
    scale_l = jnp.tile(scale_nc, (1, W)).reshape(N, 1, WCout)
    shift_l = jnp.tile(shift_nc, (1, W)).reshape(N, 1, WCout)

    b2 = 128 if N % 128 == 0 else 1
    g2 = N // b2
    out_rows = pl.pallas_call(
        _apply_kernel,
        grid=(g2,),
        in_specs=[
            pl.BlockSpec((b2, H, WCout), lambda n: (n, 0, 0)),
            pl.BlockSpec((b2, 1, WCout), lambda n: (n, 0, 0)),
            pl.BlockSpec((b2, 1, WCout), lambda n: (n, 0, 0)),
        ],
        out_specs=pl.BlockSpec((b2, H, WCout), lambda n: (n, 0, 0)),
        out_shape=jax.ShapeDtypeStruct((N, H, WCout), jnp.float32),
        compiler_params=cparams,
    )(y_rows, scale_l, shift_l)

    return jnp.transpose(out_rows.reshape(N, H, W, Cout), (0, 3, 1, 2))


# speedup vs baseline: 1.0551x; 1.0551x over previous
import functools

import jax
import jax.numpy as jnp
from jax import lax
from jax.experimental import pallas as pl
from jax.experimental.pallas import tpu as pltpu

_EPS = 1e-5


def _conv_stats_kernel(x_ref, band_ref, y_ref, stats_ref):
    B, H, WCin = x_ref.shape
    x = x_ref[...].astype(jnp.bfloat16)
    z = jnp.zeros((B, 1, WCin), jnp.bfloat16)
    x = jnp.concatenate([z, x, z], axis=1)
    acc = jnp.dot(x[:, 0:H, :].reshape(B * H, WCin), band_ref[0],
                  preferred_element_type=jnp.float32)
    acc = acc + jnp.dot(x[:, 1:H + 1, :].reshape(B * H, WCin), band_ref[1],
                        preferred_element_type=jnp.float32)
    acc = acc + jnp.dot(x[:, 2:H + 2, :].reshape(B * H, WCin), band_ref[2],
                        preferred_element_type=jnp.float32)
    s1 = jnp.sum(acc, axis=0, keepdims=True)
    s2 = jnp.sum(acc * acc, axis=0, keepdims=True)
    stats_ref[0] = jnp.concatenate([s1, s2], axis=0)
    y_ref[...] = acc.reshape(B, H, -1).astype(jnp.bfloat16)


def _apply_kernel(stats_ref, gb_ref, mask_ref, y_ref, o_ref, *, cnt):
    B, H, WCout = y_ref.shape
    Cout = gb_ref.shape[1]
    W = WCout // Cout

    st = jnp.sum(stats_ref[...], axis=0)
    lane = lax.broadcasted_iota(jnp.int32, (WCout, Cout), 0)
    co = lax.broadcasted_iota(jnp.int32, (WCout, Cout), 1)
    red = (lane % Cout == co).astype(jnp.float32)
    s = jnp.dot(st, red, preferred_element_type=jnp.float32)
    mean = s[0:1] * (1.0 / cnt)
    var = jnp.maximum(s[1:2] * (1.0 / cnt) - mean * mean, 0.0)
    inv_std = lax.rsqrt(var + _EPS)
    scale_c = gb_ref[0:1] * inv_std
    shift_c = gb_ref[1:2] - mean * scale_c
    m = mask_ref[...]
    scale_nc = scale_c * m
    shift_nc = shift_c * m
    scale_l = jnp.dot(scale_nc, red.T, preferred_element_type=jnp.float32)
    shift_l = jnp.dot(shift_nc, red.T, preferred_element_type=jnp.float32)
    y = y_ref[...].astype(jnp.float32)
    o_ref[...] = jnp.maximum(y * scale_l[:, None, :] + shift_l[:, None, :],
                             0.0)


def kernel(x_nchw, bands, b, gamma, beta, drop_mask_nc):
    del b
    N, Cin, H, W = x_nchw.shape
    WCin = W * Cin
    WCout = bands.shape[-1]
    Cout = WCout // W

    x_rows = jnp.transpose(x_nchw, (0, 2, 3, 1)).reshape(N, H, WCin)
    bands_bf = bands.astype(jnp.bfloat16)
    gb = jnp.stack([gamma, beta], axis=0)

    cparams = pltpu.CompilerParams(
        dimension_semantics=("parallel",),
        vmem_limit_bytes=64 * 1024 * 1024,
    )

    b1 = 64 if N % 64 == 0 else 1
    g1 = N // b1
    y_rows, stats = pl.pallas_call(
        _conv_stats_kernel,
        grid=(g1,),
        in_specs=[
            pl.BlockSpec((b1, H, WCin), lambda n: (n, 0, 0)),
            pl.BlockSpec((3, WCin, WCout), lambda n: (0, 0, 0)),
        ],
        out_specs=[
            pl.BlockSpec((b1, H, WCout), lambda n: (n, 0, 0)),
            pl.BlockSpec((1, 2, WCout), lambda n: (n, 0, 0)),
        ],
        out_shape=[
            jax.ShapeDtypeStruct((N, H, WCout), jnp.bfloat16),
            jax.ShapeDtypeStruct((g1, 2, WCout), jnp.float32),
        ],
        compiler_params=cparams,
    )(x_rows, bands_bf)

    b2 = 128 if N % 128 == 0 else 1
    g2 = N // b2
    out_rows = pl.pallas_call(
        functools.partial(_apply_kernel, cnt=float(N * H * W)),
        grid=(g2,),
        in_specs=[
            pl.BlockSpec((g1, 2, WCout), lambda n: (0, 0, 0)),
            pl.BlockSpec((2, Cout), lambda n: (0, 0)),
            pl.BlockSpec((b2, Cout), lambda n: (n, 0)),
            pl.BlockSpec((b2, H, WCout), lambda n: (n, 0, 0)),
        ],
        out_specs=pl.BlockSpec((b2, H, WCout), lambda n: (n, 0, 0)),
        out_shape=jax.ShapeDtypeStruct((N, H, WCout), jnp.float32),
        compiler_params=cparams,
    )(stats, gb, drop_mask_nc, y_rows)

    return jnp.transpose(out_rows.reshape(N, H, W, Cout), (0, 3, 1, 2))


# speedup vs baseline: 1.0639x; 1.0083x over previous
import functools

import jax
import jax.numpy as jnp
from jax import lax
from jax.experimental import pallas as pl
from jax.experimental.pallas import tpu as pltpu

_EPS = 1e-5


def _conv_stats_kernel(x_ref, band_ref, y_ref, stats_ref):
    B, H, WCin = x_ref.shape
    x = x_ref[...].astype(jnp.bfloat16)
    z = jnp.zeros((B, 1, WCin), jnp.bfloat16)
    x = jnp.concatenate([z, x, z], axis=1)
    acc = jnp.dot(x[:, 0:H, :].reshape(B * H, WCin), band_ref[0],
                  preferred_element_type=jnp.float32)
    acc = acc + jnp.dot(x[:, 1:H + 1, :].reshape(B * H, WCin), band_ref[1],
                        preferred_element_type=jnp.float32)
    acc = acc + jnp.dot(x[:, 2:H + 2, :].reshape(B * H, WCin), band_ref[2],
                        preferred_element_type=jnp.float32)
    s1 = jnp.sum(acc, axis=0, keepdims=True)
    s2 = jnp.sum(acc * acc, axis=0, keepdims=True)
    stats_ref[0] = jnp.concatenate([s1, s2], axis=0)
    y_ref[...] = acc.reshape(B, H, -1).astype(jnp.bfloat16)


def _apply_kernel(stats_ref, gb_ref, mask_ref, y_ref, o_ref, *, cnt):
    B, H, WCout = y_ref.shape
    Cout = gb_ref.shape[1]
    W = WCout // Cout

    st = jnp.sum(stats_ref[...], axis=0)
    lane = lax.broadcasted_iota(jnp.int32, (WCout, Cout), 0)
    co = lax.broadcasted_iota(jnp.int32, (WCout, Cout), 1)
    red = (lane % Cout == co).astype(jnp.float32)
    s = jnp.dot(st, red, preferred_element_type=jnp.float32)
    mean = s[0:1] * (1.0 / cnt)
    var = jnp.maximum(s[1:2] * (1.0 / cnt) - mean * mean, 0.0)
    inv_std = lax.rsqrt(var + _EPS)
    scale_c = gb_ref[0:1] * inv_std
    shift_c = gb_ref[1:2] - mean * scale_c
    m = mask_ref[...]
    scale_nc = scale_c * m
    shift_nc = shift_c * m
    scale_l = jnp.dot(scale_nc, red.T, preferred_element_type=jnp.float32)
    shift_l = jnp.dot(shift_nc, red.T, preferred_element_type=jnp.float32)
    y = y_ref[...].astype(jnp.float32)
    o_ref[...] = jnp.maximum(y * scale_l[:, None, :] + shift_l[:, None, :],
                             0.0)


def kernel(x_nchw, bands, b, gamma, beta, drop_mask_nc):
    del b
    N, Cin, H, W = x_nchw.shape
    WCin = W * Cin
    WCout = bands.shape[-1]
    Cout = WCout // W

    x_rows = jnp.transpose(x_nchw, (0, 2, 3, 1)).reshape(N, H, WCin)
    bands_bf = bands.astype(jnp.bfloat16)
    gb = jnp.stack([gamma, beta], axis=0)

    cparams = pltpu.CompilerParams(
        dimension_semantics=("parallel",),
        vmem_limit_bytes=64 * 1024 * 1024,
    )

    b1 = 128 if N % 128 == 0 else 1
    g1 = N // b1
    y_rows, stats = pl.pallas_call(
        _conv_stats_kernel,
        grid=(g1,),
        in_specs=[
            pl.BlockSpec((b1, H, WCin), lambda n: (n, 0, 0)),
            pl.BlockSpec((3, WCin, WCout), lambda n: (0, 0, 0)),
        ],
        out_specs=[
            pl.BlockSpec((b1, H, WCout), lambda n: (n, 0, 0)),
            pl.BlockSpec((1, 2, WCout), lambda n: (n, 0, 0)),
        ],
        out_shape=[
            jax.ShapeDtypeStruct((N, H, WCout), jnp.bfloat16),
            jax.ShapeDtypeStruct((g1, 2, WCout), jnp.float32),
        ],
        compiler_params=cparams,
    )(x_rows, bands_bf)

    b2 = 256 if N % 256 == 0 else 1
    g2 = N // b2
    out_rows = pl.pallas_call(
        functools.partial(_apply_kernel, cnt=float(N * H * W)),
        grid=(g2,),
        in_specs=[
            pl.BlockSpec((g1, 2, WCout), lambda n: (0, 0, 0)),
            pl.BlockSpec((2, Cout), lambda n: (0, 0)),
            pl.BlockSpec((b2, Cout), lambda n: (n, 0)),
            pl.BlockSpec((b2, H, WCout), lambda n: (n, 0, 0)),
        ],
        out_specs=pl.BlockSpec((b2, H, WCout), lambda n: (n, 0, 0)),
        out_shape=jax.ShapeDtypeStruct((N, H, WCout), jnp.float32),
        compiler_params=cparams,
    )(stats, gb, drop_mask_nc, y_rows)

    return jnp.transpose(out_rows.reshape(N, H, W, Cout), (0, 3, 1, 2))
